# Initial kernel scaffold; baseline (speedup 1.0000x reference)
import jax
import jax.numpy as jnp
from jax.experimental import pallas as pl


def kernel(x, edge_index, W1, b1, W2, b2, w_ih_f, w_hh_f, b_ih_f, b_hh_f, w_ih_b, w_hh_b, b_ih_b, b_hh_b, W_att, b_att, W3, b3):
    raise NotImplementedError("write your pallas kernel here")


# speedup vs baseline: 19.1017x; 19.1017x over previous
import functools

import jax
import jax.numpy as jnp
from jax import lax
from jax.experimental import pallas as pl
from jax.experimental.pallas import tpu as pltpu
from jax.experimental.pallas import tpu_sc as plsc

N = 10000
NP = 10112
E_RAW = 320000
E_TOT = E_RAW + N
C = 128
NS = 16
NCORES = 2
E_PAD = ((E_TOT + (C * NS * NCORES) - 1) // (C * NS * NCORES)) * (C * NS * NCORES)
F_IN = 128
HID = 16
N_CLASSES = 40
RPT = NP // NS


def _sc_propagate_builder(compute_dis: bool):
    mesh = plsc.VectorSubcoreMesh(core_axis_name="c", subcore_axis_name="s")
    e_per_tile16 = E_PAD // NS
    chunks16 = e_per_tile16 // C
    e_per_tile32 = E_PAD // (NS * NCORES)
    chunks32 = e_per_tile32 // C

    if compute_dis:
        out_type = [jax.ShapeDtypeStruct((NCORES, NP, HID), jnp.float32),
                    jax.ShapeDtypeStruct((NP, HID), jnp.float32)]
    else:
        out_type = jax.ShapeDtypeStruct((NCORES, NP, HID), jnp.float32)

    scratch = [
        pltpu.VMEM((C,), jnp.int32),
        pltpu.VMEM((C,), jnp.int32),
        pltpu.VMEM((C, HID), jnp.float32),
        pltpu.VMEM((RPT, HID), jnp.float32),
        pltpu.VMEM((RPT, HID), jnp.float32),
        pltpu.VMEM((RPT, HID), jnp.float32),
        pltpu.VMEM_SHARED((NP, HID), jnp.float32),
        pltpu.VMEM_SHARED((NP, HID), jnp.float32),
        pltpu.SemaphoreType.DMA,
    ]

    @functools.partial(pl.kernel, mesh=mesh, out_type=out_type,
                       scratch_types=scratch,
                       compiler_params=pltpu.CompilerParams(
                           use_tc_tiling_on_sc=False))
    def prop(*refs):
        if compute_dis:
            (row_h, col_h, y_h, out_h, dis_h,
             ridx, cidx, buf, dbuf, ybuf, zbuf, z_sh, acc_sh, sem) = refs
        else:
            (row_h, col_h, y_h, dis_in_h, out_h,
             ridx, cidx, buf, dbuf, ybuf, zbuf, z_sh, acc_sh, sem) = refs

        cid = lax.axis_index("c")
        sid = lax.axis_index("s")
        wid = sid * NCORES + cid
        node_base = sid * RPT

        def _zero_row(i, _):
            zbuf[i] = jnp.zeros((HID,), jnp.float32)
            return 0
        lax.fori_loop(0, RPT, _zero_row, 0)

        if compute_dis:
            pltpu.sync_copy(zbuf, acc_sh.at[pl.ds(node_base, RPT)])

            def _ones_row(i, _):
                buf[i] = jnp.ones((HID,), jnp.float32)
                return 0
            lax.fori_loop(0, C, _ones_row, 0)
            plsc.subcore_barrier()

            def _deg_chunk(k, _):
                base = sid * e_per_tile16 + k * C
                pltpu.sync_copy(col_h.at[pl.ds(base, C)], cidx)
                pltpu.sync_copy(buf, acc_sh.at[cidx], add=True)
                return 0
            lax.fori_loop(0, chunks16, _deg_chunk, 0)
            plsc.subcore_barrier()

            pltpu.sync_copy(acc_sh.at[pl.ds(node_base, RPT)], dbuf)

            def _newton(i, _):
                x = dbuf[i]
                xi = lax.bitcast_convert_type(x, jnp.int32)
                yi = jnp.full((HID,), 0x5F3759DF, jnp.int32) - lax.shift_right_logical(
                    xi, jnp.full((HID,), 1, jnp.int32))
                y = lax.bitcast_convert_type(yi, jnp.float32)
                hx = x * jnp.full((HID,), 0.5, jnp.float32)
                for _it in range(3):
                    y = y * (jnp.full((HID,), 1.5, jnp.float32) - hx * y * y)
                dbuf[i] = y
                return 0
            lax.fori_loop(0, RPT, _newton, 0)

            @pl.when(cid == 0)
            def _():
                pltpu.sync_copy(dbuf, dis_h.at[pl.ds(node_base, RPT)])
            plsc.subcore_barrier()
        else:
            pltpu.sync_copy(dis_in_h.at[pl.ds(node_base, RPT)], dbuf)

        pltpu.sync_copy(y_h.at[pl.ds(node_base, RPT)], ybuf)

        def _scale_row(i, _):
            ybuf[i] = ybuf[i] * dbuf[i]
            return 0
        lax.fori_loop(0, RPT, _scale_row, 0)
        pltpu.sync_copy(ybuf, z_sh.at[pl.ds(node_base, RPT)])
        pltpu.sync_copy(zbuf, acc_sh.at[pl.ds(node_base, RPT)])
        plsc.subcore_barrier()

        def _edge_chunk(k, _):
            base = wid * e_per_tile32 + k * C
            pltpu.sync_copy(row_h.at[pl.ds(base, C)], ridx)
            pltpu.async_copy(z_sh.at[ridx], buf, sem).wait()
            pltpu.sync_copy(col_h.at[pl.ds(base, C)], cidx)
            pltpu.sync_copy(buf, acc_sh.at[cidx], add=True)
            return 0
        lax.fori_loop(0, chunks32, _edge_chunk, 0)
        plsc.subcore_barrier()

        pltpu.sync_copy(acc_sh.at[pl.ds(node_base, RPT)], ybuf)
        lax.fori_loop(0, RPT, _scale_row, 0)
        pltpu.sync_copy(ybuf, out_h.at[cid].at[pl.ds(node_base, RPT)])

    return prop


_prop_first = _sc_propagate_builder(compute_dis=True)
_prop_next = _sc_propagate_builder(compute_dis=False)


def _tc_mm1(x, w1):
    def body(x_ref, w_ref, o_ref):
        o_ref[...] = lax.dot_general(
            x_ref[...], w_ref[...], (((1,), (1,)), ((), ())),
            preferred_element_type=jnp.float32)
    return pl.pallas_call(
        body, out_shape=jax.ShapeDtypeStruct((NP, HID), jnp.float32))(x, w1)


def _tc_mid1(parts, b1, w2):
    def body(p_ref, b_ref, w_ref, h_ref, g_ref):
        h1 = jnp.maximum(p_ref[0] + p_ref[1] + b_ref[...], 0.0)
        h_ref[...] = h1
        g_ref[...] = lax.dot_general(
            h1, w_ref[...], (((1,), (1,)), ((), ())),
            preferred_element_type=jnp.float32)
    return pl.pallas_call(
        body, out_shape=[jax.ShapeDtypeStruct((NP, HID), jnp.float32),
                         jax.ShapeDtypeStruct((NP, HID), jnp.float32)])(
            parts, b1, w2)


def _tc_mid2(parts2, b2, h1, w_ih_f, w_hh_f, b_ih_f, b_hh_f,
             w_ih_b, w_hh_b, b_ih_b, b_hh_b, w_att, b_att):
    def cell(x_t, h, c, w_ih, w_hh, b_ih, b_hh):
        gates = lax.dot_general(x_t, w_ih, (((1,), (1,)), ((), ())),
                                preferred_element_type=jnp.float32) + b_ih
        if h is not None:
            gates = gates + lax.dot_general(
                h, w_hh, (((1,), (1,)), ((), ())),
                preferred_element_type=jnp.float32)
        gates = gates + b_hh
        i_g = jax.nn.sigmoid(gates[:, 0:32])
        f_g = jax.nn.sigmoid(gates[:, 32:64])
        g_g = jnp.tanh(gates[:, 64:96])
        o_g = jax.nn.sigmoid(gates[:, 96:128])
        c_new = (f_g * c if c is not None else 0.0) + i_g * g_g
        return o_g * jnp.tanh(c_new), c_new

    def body(p_ref, b2_ref, h1_ref, wif_ref, whf_ref, bif_ref, bhf_ref,
             wib_ref, whb_ref, bib_ref, bhb_ref, wa_ref, ba_ref, xj_ref):
        h1 = h1_ref[...]
        h2 = jnp.maximum(p_ref[0] + p_ref[1] + b2_ref[...], 0.0)
        wif, whf, bif, bhf = wif_ref[...], whf_ref[...], bif_ref[...], bhf_ref[...]
        wib, whb, bib, bhb = wib_ref[...], whb_ref[...], bib_ref[...], bhb_ref[...]
        hf0, cf0 = cell(h1, None, None, wif, whf, bif, bhf)
        hf1, _ = cell(h2, hf0, cf0, wif, whf, bif, bhf)
        hb1, cb1 = cell(h2, None, None, wib, whb, bib, bhb)
        hb0, _ = cell(h1, hb1, cb1, wib, whb, bib, bhb)
        wa = wa_ref[...]
        waf, wab = wa[:, :32], wa[:, 32:]
        ba = ba_ref[0, 0]
        a0 = (jnp.sum(hf0 * waf, axis=1) + jnp.sum(hb0 * wab, axis=1)) + ba
        a1 = (jnp.sum(hf1 * waf, axis=1) + jnp.sum(hb1 * wab, axis=1)) + ba
        m = jnp.maximum(a0, a1)
        e0 = jnp.exp(a0 - m)
        e1 = jnp.exp(a1 - m)
        s = e0 + e1
        xj_ref[...] = (e0 / s)[:, None] * h1 + (e1 / s)[:, None] * h2

    grid = 8
    rb = NP // grid
    full = lambda shape: pl.BlockSpec(shape, lambda i: (0,) * len(shape))
    return pl.pallas_call(
        body,
        grid=(grid,),
        in_specs=[
            pl.BlockSpec((2, rb, HID), lambda i: (0, i, 0)),
            full((1, HID)),
            pl.BlockSpec((rb, HID), lambda i: (i, 0)),
            full((128, HID)), full((128, 32)), full((1, 128)), full((1, 128)),
            full((128, HID)), full((128, 32)), full((1, 128)), full((1, 128)),
            full((1, 64)), full((1, 1)),
        ],
        out_specs=pl.BlockSpec((rb, HID), lambda i: (i, 0)),
        out_shape=jax.ShapeDtypeStruct((NP, HID), jnp.float32))(
            parts2, b2, h1, w_ih_f, w_hh_f, b_ih_f, b_hh_f,
            w_ih_b, w_hh_b, b_ih_b, b_hh_b, w_att, b_att)


def _tc_final(parts3, w3, b3):
    def body(p_ref, w_ref, b_ref, o_ref):
        xp = p_ref[0] + p_ref[1]
        logits = lax.dot_general(
            xp, w_ref[...], (((1,), (1,)), ((), ())),
            preferred_element_type=jnp.float32) + b_ref[...]
        m = jnp.max(logits, axis=1, keepdims=True)
        lo = logits - m
        o_ref[...] = lo - jnp.log(jnp.sum(jnp.exp(lo), axis=1, keepdims=True))
    return pl.pallas_call(
        body, out_shape=jax.ShapeDtypeStruct((NP, N_CLASSES), jnp.float32))(
            parts3, w3, b3)


def kernel(x, edge_index, W1, b1, W2, b2, w_ih_f, w_hh_f, b_ih_f, b_hh_f,
           w_ih_b, w_hh_b, b_ih_b, b_hh_b, W_att, b_att, W3, b3):
    x = jnp.pad(x, ((0, NP - N), (0, 0)))
    loops = jnp.arange(N, dtype=jnp.int32)
    n_fill = E_PAD - E_TOT
    fill = jnp.arange(n_fill, dtype=jnp.int32)
    fill_row = (fill * 37) % N
    fill_col = N + (fill % (NP - N))
    row = jnp.concatenate([edge_index[0], loops, fill_row]).astype(jnp.int32)
    col = jnp.concatenate([edge_index[1], loops, fill_col]).astype(jnp.int32)

    b1r = b1.reshape(1, HID)
    b2r = b2.reshape(1, HID)
    bif = b_ih_f.reshape(1, 128)
    bhf = b_hh_f.reshape(1, 128)
    bib = b_ih_b.reshape(1, 128)
    bhb = b_hh_b.reshape(1, 128)
    bar = b_att.reshape(1, 1)
    b3r = b3.reshape(1, N_CLASSES)

    y1 = _tc_mm1(x, W1)
    p1, dis2d = _prop_first(row, col, y1)
    h1, g2 = _tc_mid1(p1, b1r, W2)
    p2 = _prop_next(row, col, g2, dis2d)
    xj = _tc_mid2(p2, b2r, h1, w_ih_f, w_hh_f, bif, bhf,
                  w_ih_b, w_hh_b, bib, bhb, W_att, bar)
    p3 = _prop_next(row, col, xj, dis2d)
    out = _tc_final(p3, W3, b3r)
    return out[:N]


# speedup vs baseline: 41.0851x; 2.1509x over previous
import functools

import jax
import jax.numpy as jnp
from jax import lax
from jax.experimental import pallas as pl
from jax.experimental.pallas import tpu as pltpu
from jax.experimental.pallas import tpu_sc as plsc

N = 10000
NP = 10112
E_RAW = 320000
E_TOT = E_RAW + N
C = 128
NS = 16
NCORES = 2
_BLOCKS = (E_TOT + (C * NS * NCORES) - 1) // (C * NS * NCORES)
_BLOCKS += _BLOCKS % 2
E_PAD = _BLOCKS * C * NS * NCORES
F_IN = 128
HID = 16
N_CLASSES = 40
RPT = NP // NS


def _sc_propagate_builder(compute_dis: bool):
    mesh = plsc.VectorSubcoreMesh(core_axis_name="c", subcore_axis_name="s")
    chunks16 = E_PAD // (NS * C)
    chunks32 = E_PAD // (NS * NCORES * C)

    if compute_dis:
        out_type = [jax.ShapeDtypeStruct((NCORES, NP, HID), jnp.float32),
                    jax.ShapeDtypeStruct((NP, HID), jnp.float32)]
    else:
        out_type = jax.ShapeDtypeStruct((NCORES, NP, HID), jnp.float32)

    scratch = [
        pltpu.VMEM((chunks32, C), jnp.int32),
        pltpu.VMEM((chunks32, C), jnp.int32),
        pltpu.VMEM((2, C, HID), jnp.float32),
        pltpu.VMEM((RPT, HID), jnp.float32),
        pltpu.VMEM((RPT, HID), jnp.float32),
        pltpu.VMEM((RPT, HID), jnp.float32),
        pltpu.VMEM_SHARED((NP, HID), jnp.float32),
        pltpu.VMEM_SHARED((NP, HID), jnp.float32),
        pltpu.SemaphoreType.DMA,
        pltpu.SemaphoreType.DMA,
        pltpu.SemaphoreType.DMA,
    ]
    if compute_dis:
        scratch.append(pltpu.VMEM((chunks16, C), jnp.int32))

    @functools.partial(pl.kernel, mesh=mesh, out_type=out_type,
                       scratch_types=scratch,
                       compiler_params=pltpu.CompilerParams(
                           use_tc_tiling_on_sc=False))
    def prop(*refs):
        if compute_dis:
            (row_h, col_h, y_h, out_h, dis_h,
             ridx_all, cidx_all, buf, dbuf, ybuf, zbuf, z_sh, acc_sh,
             gsem0, gsem1, ssem, degidx) = refs
        else:
            (row_h, col_h, y_h, dis_in_h, out_h,
             ridx_all, cidx_all, buf, dbuf, ybuf, zbuf, z_sh, acc_sh,
             gsem0, gsem1, ssem) = refs

        cid = lax.axis_index("c")
        sid = lax.axis_index("s")
        wid = sid * NCORES + cid
        node_base = sid * RPT

        def _zero_row(i, _):
            zbuf[i] = jnp.zeros((HID,), jnp.float32)
            return 0
        lax.fori_loop(0, RPT, _zero_row, 0)

        if compute_dis:
            pltpu.sync_copy(zbuf, acc_sh.at[pl.ds(node_base, RPT)])
            pltpu.sync_copy(col_h.at[pl.ds(sid * chunks16, chunks16)], degidx)

            ones = ybuf.at[pl.ds(0, C)]

            def _ones_row(i, _):
                ybuf[i] = jnp.ones((HID,), jnp.float32)
                return 0
            lax.fori_loop(0, C, _ones_row, 0)
            plsc.subcore_barrier()

            def _deg_group(g, _):
                for j in range(4):
                    pltpu.async_copy(ones, acc_sh.at[degidx.at[g * 4 + j]],
                                     ssem, add=True)
                for j in range(4):
                    pltpu.make_async_copy(
                        ones, acc_sh.at[degidx.at[g * 4 + j]], ssem).wait()
                return 0
            lax.fori_loop(0, chunks16 // 4, _deg_group, 0)
            plsc.subcore_barrier()

            pltpu.sync_copy(acc_sh.at[pl.ds(node_base, RPT)], dbuf)

            def _newton(i, _):
                x = dbuf[i]
                xi = lax.bitcast_convert_type(x, jnp.int32)
                yi = jnp.full((HID,), 0x5F3759DF, jnp.int32) - lax.shift_right_logical(
                    xi, jnp.full((HID,), 1, jnp.int32))
                y = lax.bitcast_convert_type(yi, jnp.float32)
                hx = x * jnp.full((HID,), 0.5, jnp.float32)
                for _it in range(3):
                    y = y * (jnp.full((HID,), 1.5, jnp.float32) - hx * y * y)
                dbuf[i] = y
                return 0
            lax.fori_loop(0, RPT, _newton, 0)

            @pl.when(cid == 0)
            def _():
                pltpu.sync_copy(dbuf, dis_h.at[pl.ds(node_base, RPT)])
            plsc.subcore_barrier()
        else:
            pltpu.sync_copy(dis_in_h.at[pl.ds(node_base, RPT)], dbuf)

        pltpu.sync_copy(y_h.at[pl.ds(node_base, RPT)], ybuf)

        def _scale_row(i, _):
            ybuf[i] = ybuf[i] * dbuf[i]
            return 0
        lax.fori_loop(0, RPT, _scale_row, 0)
        pltpu.sync_copy(ybuf, z_sh.at[pl.ds(node_base, RPT)])
        pltpu.sync_copy(zbuf, acc_sh.at[pl.ds(node_base, RPT)])
        plsc.subcore_barrier()

        cbase = wid * chunks32
        pltpu.sync_copy(row_h.at[pl.ds(cbase, chunks32)], ridx_all)
        pltpu.sync_copy(col_h.at[pl.ds(cbase, chunks32)], cidx_all)
        buf0, buf1 = buf.at[0], buf.at[1]
        npairs = chunks32 // 2

        pltpu.async_copy(z_sh.at[ridx_all.at[0]], buf0, gsem0)

        def _edge_pair(p, _):
            k0 = p * 2
            pltpu.make_async_copy(z_sh.at[ridx_all.at[k0]], buf0, gsem0).wait()
            pltpu.async_copy(z_sh.at[ridx_all.at[k0 + 1]], buf1, gsem1)
            pltpu.sync_copy(buf0, acc_sh.at[cidx_all.at[k0]], add=True)
            pltpu.make_async_copy(
                z_sh.at[ridx_all.at[k0 + 1]], buf1, gsem1).wait()

            @pl.when(p + 1 < npairs)
            def _():
                pltpu.async_copy(z_sh.at[ridx_all.at[k0 + 2]], buf0, gsem0)
            pltpu.sync_copy(buf1, acc_sh.at[cidx_all.at[k0 + 1]], add=True)
            return 0
        lax.fori_loop(0, npairs, _edge_pair, 0)
        plsc.subcore_barrier()

        pltpu.sync_copy(acc_sh.at[pl.ds(node_base, RPT)], ybuf)
        lax.fori_loop(0, RPT, _scale_row, 0)
        pltpu.sync_copy(ybuf, out_h.at[cid].at[pl.ds(node_base, RPT)])

    return prop


_prop_first = _sc_propagate_builder(compute_dis=True)
_prop_next = _sc_propagate_builder(compute_dis=False)


def _tc_mm1(x, w1):
    def body(x_ref, w_ref, o_ref):
        o_ref[...] = lax.dot_general(
            x_ref[...], w_ref[...], (((1,), (1,)), ((), ())),
            preferred_element_type=jnp.float32)
    return pl.pallas_call(
        body, out_shape=jax.ShapeDtypeStruct((NP, HID), jnp.float32))(x, w1)


def _tc_mid1(parts, b1, w2):
    def body(p_ref, b_ref, w_ref, h_ref, g_ref):
        h1 = jnp.maximum(p_ref[0] + p_ref[1] + b_ref[...], 0.0)
        h_ref[...] = h1
        g_ref[...] = lax.dot_general(
            h1, w_ref[...], (((1,), (1,)), ((), ())),
            preferred_element_type=jnp.float32)
    return pl.pallas_call(
        body, out_shape=[jax.ShapeDtypeStruct((NP, HID), jnp.float32),
                         jax.ShapeDtypeStruct((NP, HID), jnp.float32)])(
            parts, b1, w2)


def _tc_mid2(parts2, b2, h1, w_ih_f, w_hh_f, b_ih_f, b_hh_f,
             w_ih_b, w_hh_b, b_ih_b, b_hh_b, w_att, b_att):
    def cell(x_t, h, c, w_ih, w_hh, b_ih, b_hh):
        gates = lax.dot_general(x_t, w_ih, (((1,), (1,)), ((), ())),
                                preferred_element_type=jnp.float32) + b_ih
        if h is not None:
            gates = gates + lax.dot_general(
                h, w_hh, (((1,), (1,)), ((), ())),
                preferred_element_type=jnp.float32)
        gates = gates + b_hh
        i_g = jax.nn.sigmoid(gates[:, 0:32])
        f_g = jax.nn.sigmoid(gates[:, 32:64])
        g_g = jnp.tanh(gates[:, 64:96])
        o_g = jax.nn.sigmoid(gates[:, 96:128])
        c_new = (f_g * c if c is not None else 0.0) + i_g * g_g
        return o_g * jnp.tanh(c_new), c_new

    def body(p_ref, b2_ref, h1_ref, wif_ref, whf_ref, bif_ref, bhf_ref,
             wib_ref, whb_ref, bib_ref, bhb_ref, wa_ref, ba_ref, xj_ref):
        h1 = h1_ref[...]
        h2 = jnp.maximum(p_ref[0] + p_ref[1] + b2_ref[...], 0.0)
        wif, whf, bif, bhf = wif_ref[...], whf_ref[...], bif_ref[...], bhf_ref[...]
        wib, whb, bib, bhb = wib_ref[...], whb_ref[...], bib_ref[...], bhb_ref[...]
        hf0, cf0 = cell(h1, None, None, wif, whf, bif, bhf)
        hf1, _ = cell(h2, hf0, cf0, wif, whf, bif, bhf)
        hb1, cb1 = cell(h2, None, None, wib, whb, bib, bhb)
        hb0, _ = cell(h1, hb1, cb1, wib, whb, bib, bhb)
        wa = wa_ref[...]
        waf, wab = wa[:, :32], wa[:, 32:]
        ba = ba_ref[0, 0]
        a0 = (jnp.sum(hf0 * waf, axis=1) + jnp.sum(hb0 * wab, axis=1)) + ba
        a1 = (jnp.sum(hf1 * waf, axis=1) + jnp.sum(hb1 * wab, axis=1)) + ba
        m = jnp.maximum(a0, a1)
        e0 = jnp.exp(a0 - m)
        e1 = jnp.exp(a1 - m)
        s = e0 + e1
        xj_ref[...] = (e0 / s)[:, None] * h1 + (e1 / s)[:, None] * h2

    grid = 8
    rb = NP // grid
    full = lambda shape: pl.BlockSpec(shape, lambda i: (0,) * len(shape))
    return pl.pallas_call(
        body,
        grid=(grid,),
        in_specs=[
            pl.BlockSpec((2, rb, HID), lambda i: (0, i, 0)),
            full((1, HID)),
            pl.BlockSpec((rb, HID), lambda i: (i, 0)),
            full((128, HID)), full((128, 32)), full((1, 128)), full((1, 128)),
            full((128, HID)), full((128, 32)), full((1, 128)), full((1, 128)),
            full((1, 64)), full((1, 1)),
        ],
        out_specs=pl.BlockSpec((rb, HID), lambda i: (i, 0)),
        out_shape=jax.ShapeDtypeStruct((NP, HID), jnp.float32))(
            parts2, b2, h1, w_ih_f, w_hh_f, b_ih_f, b_hh_f,
            w_ih_b, w_hh_b, b_ih_b, b_hh_b, w_att, b_att)


def _tc_final(parts3, w3, b3):
    def body(p_ref, w_ref, b_ref, o_ref):
        xp = p_ref[0] + p_ref[1]
        logits = lax.dot_general(
            xp, w_ref[...], (((1,), (1,)), ((), ())),
            preferred_element_type=jnp.float32) + b_ref[...]
        m = jnp.max(logits, axis=1, keepdims=True)
        lo = logits - m
        o_ref[...] = lo - jnp.log(jnp.sum(jnp.exp(lo), axis=1, keepdims=True))
    return pl.pallas_call(
        body, out_shape=jax.ShapeDtypeStruct((NP, N_CLASSES), jnp.float32))(
            parts3, w3, b3)


def kernel(x, edge_index, W1, b1, W2, b2, w_ih_f, w_hh_f, b_ih_f, b_hh_f,
           w_ih_b, w_hh_b, b_ih_b, b_hh_b, W_att, b_att, W3, b3):
    x = jnp.pad(x, ((0, NP - N), (0, 0)))
    loops = jnp.arange(N, dtype=jnp.int32)
    n_fill = E_PAD - E_TOT
    fill = jnp.arange(n_fill, dtype=jnp.int32)
    fill_row = (fill * 37) % N
    fill_col = N + (fill % (NP - N))
    row = jnp.concatenate([edge_index[0], loops, fill_row]).astype(
        jnp.int32).reshape(E_PAD // C, C)
    col = jnp.concatenate([edge_index[1], loops, fill_col]).astype(
        jnp.int32).reshape(E_PAD // C, C)

    b1r = b1.reshape(1, HID)
    b2r = b2.reshape(1, HID)
    bif = b_ih_f.reshape(1, 128)
    bhf = b_hh_f.reshape(1, 128)
    bib = b_ih_b.reshape(1, 128)
    bhb = b_hh_b.reshape(1, 128)
    bar = b_att.reshape(1, 1)
    b3r = b3.reshape(1, N_CLASSES)

    y1 = _tc_mm1(x, W1)
    p1, dis2d = _prop_first(row, col, y1)
    h1, g2 = _tc_mid1(p1, b1r, W2)
    p2 = _prop_next(row, col, g2, dis2d)
    xj = _tc_mid2(p2, b2r, h1, w_ih_f, w_hh_f, bif, bhf,
                  w_ih_b, w_hh_b, bib, bhb, W_att, bar)
    p3 = _prop_next(row, col, xj, dis2d)
    out = _tc_final(p3, W3, b3r)
    return out[:N]
